# baseline (speedup 1.0000x reference)
import functools

import jax
import jax.numpy as jnp
from jax import lax
from jax.experimental import pallas as pl
from jax.experimental.pallas import tpu as pltpu
from jax.experimental.pallas import tpu_sc as plsc

N_EMBEDS = 1000000
EMBED_DIM = 64
BATCH = 16384
FIELDS = 26

_NC = 2
_NS = 16
_NW = _NC * _NS
_B = BATCH * FIELDS
_BPW = _B // _NW
_NBT = BATCH // 128
_BT_PER_W = _NBT // _NW
_NGROUP = _BT_PER_W * FIELDS


def _emb_kernel(idx_hbm, table_hbm, out_hbm, idx_v, sel_v, rows_v, tile_v,
                gsem, wsem):
    wid = lax.axis_index("s") * _NC + lax.axis_index("c")
    base = wid * _BPW
    pltpu.sync_copy(idx_hbm.at[pl.ds(base, _BPW)], idx_v)
    lane = lax.iota(jnp.int32, 16)

    def group_body(g, carry):
        bt_local = g // FIELDS
        f = g - bt_local * FIELDS
        ibase = bt_local * (128 * FIELDS) + f
        for m in range(8):
            iv = ibase + (m * 16 + lane) * FIELDS
            sel_v[pl.ds(m * 16, 16)] = plsc.load_gather(idx_v, [iv])
        pltpu.async_copy(table_hbm.at[sel_v], rows_v, gsem).wait()
        for d in range(EMBED_DIM):
            for kb in range(8):
                v = plsc.load_gather(rows_v, [kb * 16 + lane, d + lane * 0])
                tile_v[d // 8, d % 8, pl.ds(kb * 16, 16)] = v
        bt = wid * _BT_PER_W + bt_local
        pltpu.async_copy(tile_v, out_hbm.at[f, :, bt], wsem).wait()
        return carry

    lax.fori_loop(0, _NGROUP, group_body, 0)


_TROWS = 8192
_TGRID = (N_EMBEDS + _TROWS - 1) // _TROWS


def _transpose_body(i_ref, o_ref):
    t3 = i_ref[...].T.reshape(_TROWS // 2, 2, EMBED_DIM)
    o_ref[...] = jnp.concatenate([t3[:, 0, :], t3[:, 1, :]], axis=1)


def _tc_transpose(w_t):
    return pl.pallas_call(
        _transpose_body,
        grid=(_TGRID,),
        in_specs=[pl.BlockSpec((EMBED_DIM, _TROWS), lambda j: (0, j))],
        out_specs=pl.BlockSpec((_TROWS // 2, 128), lambda j: (j, 0)),
        out_shape=jax.ShapeDtypeStruct((N_EMBEDS // 2, 128), jnp.float32),
    )(w_t)


@jax.jit
def _run(idx_flat, weights):
    f = functools.partial(
        pl.kernel,
        mesh=plsc.VectorSubcoreMesh(core_axis_name="c", subcore_axis_name="s"),
        out_type=jax.ShapeDtypeStruct((FIELDS, 8, _NBT, 8, 128), jnp.float32),
        scratch_types=[
            pltpu.VMEM((_BPW,), jnp.int32),
            pltpu.VMEM((128,), jnp.int32),
            pltpu.VMEM((128, EMBED_DIM), jnp.float32),
            pltpu.VMEM((8, 8, 128), jnp.float32),
            pltpu.SemaphoreType.DMA,
            pltpu.SemaphoreType.DMA,
        ],
        compiler_params=pltpu.CompilerParams(use_tc_tiling_on_sc=False, needs_layout_passes=False),
    )(_emb_kernel)
    return f(idx_flat, weights)


def kernel(x, weights):
    table_lin = _tc_transpose(weights.T).reshape(N_EMBEDS, EMBED_DIM)
    out5 = _run(x.reshape(-1), table_lin)
    return jnp.transpose(out5, (2, 4, 0, 1, 3)).reshape(BATCH, FIELDS, EMBED_DIM)
